# baseline (speedup 1.0000x reference)
import functools

import jax
import jax.numpy as jnp
from jax import lax
from jax.experimental import pallas as pl
from jax.experimental.pallas import tpu as pltpu
from jax.experimental.pallas import tpu_sc as plsc

B, U, C, DK, DV = 16, 64, 1024, 64, 64
RATE = 0.01
L = 16
NW = 32
UPW = U // NW
PR = C // 2
CPR = 128
NCHUNK = PR // CPR
WPR = 64
BH = B // 2


def _unit_prologue(u, w_hbm, knT_hbm, vnT_hbm, mr_hbm, wbuf, knbuf, vnbuf,
                   retbuf, orbuf, rbuf):
    pltpu.sync_copy(w_hbm.at[u], wbuf)
    pltpu.sync_copy(knT_hbm.at[u], knbuf)
    pltpu.sync_copy(vnT_hbm.at[u], vnbuf)
    pltpu.sync_copy(mr_hbm.at[u], orbuf)

    rbv = [rbuf[b, :] for b in range(B)]

    def ret_body(g, carry):
        sl = pl.ds(g * L, L)
        wv0 = wbuf[0, sl]
        fs = wv0
        rw = wv0 * rbv[0]
        for b in range(1, B):
            wv = wbuf[b, sl]
            fs = fs + wv
            rw = rw + wv * rbv[b]
        retv = 1.0 - RATE * fs
        retbuf[sl] = retv
        orbuf[sl] = orbuf[sl] * retv + RATE * rw
        return carry

    lax.fori_loop(0, C // L, ret_body, 0)


def _chunk_compute(slot, ch, wbuf, retbuf, knbuf, vnbuf, mkbuf, mvbuf):
    c0 = ch * (2 * CPR)
    for half in range(2):
        for phase in range(2):
            kv_src = knbuf if phase == 0 else vnbuf
            mem = mkbuf if phase == 0 else mvbuf
            kv = [[kv_src[half * BH + b, pl.ds(j * L, L)] * RATE
                   for j in range(4)] for b in range(BH)]

            @plsc.parallel_loop(0, CPR // 8, unroll=2)
            def g_body(g):
                csl = pl.ds(c0 + g * L, L)
                wv = [wbuf[half * BH + b, csl] for b in range(BH)]
                retv = retbuf[csl] if half == 0 else None
                for i in range(8):
                    row = g * 8 + i
                    ws = [[jnp.full((L,), wv[b][2 * i + p]) for b in range(BH)]
                          for p in range(2)]
                    if half == 0:
                        rb2 = [jnp.full((L,), retv[2 * i + p]) for p in range(2)]
                    for jc in range(8):
                        par = jc // 4
                        dsl = pl.ds(jc * L, L)
                        mv = mem[slot, row, dsl]
                        base = mv * rb2[par] if half == 0 else mv
                        p0 = [ws[par][b] * kv[b][jc % 4] for b in range(BH)]
                        s0 = (p0[0] + p0[1]) + (p0[2] + p0[3])
                        s1 = (p0[4] + p0[5]) + (p0[6] + p0[7])
                        mem[slot, row, dsl] = base + (s0 + s1)


def _sc_body(w_hbm, knT_hbm, vnT_hbm, r_hbm, mk_hbm, mv_hbm, mr_hbm,
             ok_hbm, ov_hbm, or_hbm,
             wbuf, knbuf, vnbuf, rbuf, retbuf, orbuf, mkbuf, mvbuf,
             insem, outsem, orsem):
    wid = lax.axis_index("s") * 2 + lax.axis_index("c")

    pltpu.sync_copy(r_hbm, rbuf)

    def unit_body(iu, carry0):
        u = wid * UPW + iu
        _unit_prologue(u, w_hbm, knT_hbm, vnT_hbm, mr_hbm, wbuf, knbuf, vnbuf,
                       retbuf, orbuf, rbuf)
        or_cp = pltpu.make_async_copy(orbuf, or_hbm.at[u], orsem)
        or_cp.start()

        def in_copies(ch, slot):
            csl = pl.ds(ch * CPR, CPR)
            return [
                pltpu.make_async_copy(mk_hbm.at[u, csl], mkbuf.at[slot],
                                      insem.at[slot, 0]),
                pltpu.make_async_copy(mv_hbm.at[u, csl], mvbuf.at[slot],
                                      insem.at[slot, 1]),
            ]

        def out_copies(ch, slot):
            csl = pl.ds(ch * CPR, CPR)
            return [
                pltpu.make_async_copy(mkbuf.at[slot], ok_hbm.at[u, csl],
                                      outsem.at[slot, 0]),
                pltpu.make_async_copy(mvbuf.at[slot], ov_hbm.at[u, csl],
                                      outsem.at[slot, 1]),
            ]

        for cp in in_copies(0, 0):
            cp.start()

        def chunk_body(ch, carry1):
            slot = lax.rem(ch, 2)
            for cp in in_copies(ch, slot):
                cp.wait()

            @pl.when(ch + 1 < NCHUNK)
            def _prefetch():
                @pl.when(ch >= 1)
                def _drain_prev():
                    for cp in out_copies(ch - 1, 1 - slot):
                        cp.wait()
                for cp in in_copies(ch + 1, 1 - slot):
                    cp.start()

            _chunk_compute(slot, ch, wbuf, retbuf, knbuf, vnbuf, mkbuf, mvbuf)
            for cp in out_copies(ch, slot):
                cp.start()
            return carry1

        lax.fori_loop(0, NCHUNK, chunk_body, 0)
        for cp in out_copies(NCHUNK - 2, (NCHUNK - 2) % 2):
            cp.wait()
        for cp in out_copies(NCHUNK - 1, (NCHUNK - 1) % 2):
            cp.wait()
        or_cp.wait()
        return carry0

    lax.fori_loop(0, UPW, unit_body, 0)


def kernel(weights, key_new, value_new, reward, mem_keys, mem_values, mem_rewards):
    wT = weights.transpose(1, 0, 2)
    knT = key_new.transpose(1, 0, 2)
    vnT = value_new.transpose(1, 0, 2)
    rB = jnp.broadcast_to(reward.reshape(B, 1), (B, L))
    mk2 = mem_keys.reshape(U, PR, 128)
    mv2 = mem_values.reshape(U, PR, 128)
    mesh = plsc.VectorSubcoreMesh(core_axis_name="c", subcore_axis_name="s")
    sc = functools.partial(
        pl.kernel, mesh=mesh,
        out_type=[
            jax.ShapeDtypeStruct((U, PR, 128), jnp.float32),
            jax.ShapeDtypeStruct((U, PR, 128), jnp.float32),
            jax.ShapeDtypeStruct((U, C), jnp.float32),
        ],
        scratch_types=[
            pltpu.VMEM((B, C), jnp.float32),
            pltpu.VMEM((B, DK), jnp.float32),
            pltpu.VMEM((B, DV), jnp.float32),
            pltpu.VMEM((B, L), jnp.float32),
            pltpu.VMEM((C,), jnp.float32),
            pltpu.VMEM((C,), jnp.float32),
            pltpu.VMEM((2, CPR, 128), jnp.float32),
            pltpu.VMEM((2, CPR, 128), jnp.float32),
            pltpu.SemaphoreType.DMA((2, 2)),
            pltpu.SemaphoreType.DMA((2, 2)),
            pltpu.SemaphoreType.DMA,
        ],
    )(_sc_body)
    out_k, out_v, out_r = sc(wT, knT, vnT, rB, mk2, mv2, mem_rewards)
    return out_k.reshape(U, C, DK), out_v.reshape(U, C, DV), out_r


# speedup vs baseline: 1.0818x; 1.0818x over previous
import functools

import jax
import jax.numpy as jnp
from jax import lax
from jax.experimental import pallas as pl
from jax.experimental.pallas import tpu as pltpu
from jax.experimental.pallas import tpu_sc as plsc

B, U, C, DK, DV = 16, 64, 1024, 64, 64
RATE = 0.01
L = 16
NW = 32
UPW = U // NW
PR = C // 2
CPR = 128
NCHUNK = PR // CPR
WPR = 64
BH = B // 2


def _unit_prologue(u, w_hbm, knT_hbm, vnT_hbm, mr_hbm, wbuf, knbuf, vnbuf,
                   retbuf, orbuf, rbuf):
    pltpu.sync_copy(w_hbm.at[u], wbuf)
    pltpu.sync_copy(knT_hbm.at[u], knbuf)
    pltpu.sync_copy(vnT_hbm.at[u], vnbuf)
    pltpu.sync_copy(mr_hbm.at[u], orbuf)

    rbv = [rbuf[b, :] for b in range(B)]

    def ret_body(g, carry):
        sl = pl.ds(g * L, L)
        wv0 = wbuf[0, sl]
        fs = wv0
        rw = wv0 * rbv[0]
        for b in range(1, B):
            wv = wbuf[b, sl]
            fs = fs + wv
            rw = rw + wv * rbv[b]
        retv = 1.0 - RATE * fs
        retbuf[sl] = retv
        orbuf[sl] = orbuf[sl] * retv + RATE * rw
        return carry

    lax.fori_loop(0, C // L, ret_body, 0)


def _chunk_compute(slot, ch, wbuf, retbuf, knbuf, vnbuf, mkbuf, mvbuf):
    c0 = ch * (2 * CPR)
    for half in range(2):
        for phase in range(2):
            kv_src = knbuf if phase == 0 else vnbuf
            mem = mkbuf if phase == 0 else mvbuf
            kv = [[kv_src[half * BH + b, pl.ds(j * L, L)] * RATE
                   for j in range(4)] for b in range(BH)]

            def g_body(g, carry):
                csl = pl.ds(c0 + g * L, L)
                wv = [wbuf[half * BH + b, csl] for b in range(BH)]
                retv = retbuf[csl] if half == 0 else None
                for i in range(8):
                    row = g * 8 + i
                    ws = [[jnp.full((L,), wv[b][2 * i + p]) for b in range(BH)]
                          for p in range(2)]
                    if half == 0:
                        rb2 = [jnp.full((L,), retv[2 * i + p]) for p in range(2)]
                    for jc in range(8):
                        par = jc // 4
                        dsl = pl.ds(jc * L, L)
                        mv = mem[slot, row, dsl]
                        base = mv * rb2[par] if half == 0 else mv
                        p0 = [ws[par][b] * kv[b][jc % 4] for b in range(BH)]
                        s0 = (p0[0] + p0[1]) + (p0[2] + p0[3])
                        s1 = (p0[4] + p0[5]) + (p0[6] + p0[7])
                        mem[slot, row, dsl] = base + (s0 + s1)
                return carry

            lax.fori_loop(0, CPR // 8, g_body, 0)


def _sc_body(w_hbm, knT_hbm, vnT_hbm, r_hbm, mk_hbm, mv_hbm, mr_hbm,
             ok_hbm, ov_hbm, or_hbm,
             wbuf, knbuf, vnbuf, rbuf, retbuf, orbuf, mkbuf, mvbuf,
             insem, outsem, orsem):
    wid = lax.axis_index("s") * 2 + lax.axis_index("c")

    pltpu.sync_copy(r_hbm, rbuf)

    def unit_body(iu, carry0):
        u = wid * UPW + iu
        _unit_prologue(u, w_hbm, knT_hbm, vnT_hbm, mr_hbm, wbuf, knbuf, vnbuf,
                       retbuf, orbuf, rbuf)
        or_cp = pltpu.make_async_copy(orbuf, or_hbm.at[u], orsem)
        or_cp.start()

        def in_copies(ch, slot):
            csl = pl.ds(ch * CPR, CPR)
            return [
                pltpu.make_async_copy(mk_hbm.at[u, csl], mkbuf.at[slot],
                                      insem.at[slot, 0]),
                pltpu.make_async_copy(mv_hbm.at[u, csl], mvbuf.at[slot],
                                      insem.at[slot, 1]),
            ]

        def out_copies(ch, slot):
            csl = pl.ds(ch * CPR, CPR)
            return [
                pltpu.make_async_copy(mkbuf.at[slot], ok_hbm.at[u, csl],
                                      outsem.at[slot, 0]),
                pltpu.make_async_copy(mvbuf.at[slot], ov_hbm.at[u, csl],
                                      outsem.at[slot, 1]),
            ]

        for cp in in_copies(0, 0):
            cp.start()

        def chunk_body(ch, carry1):
            slot = lax.rem(ch, 2)
            for cp in in_copies(ch, slot):
                cp.wait()

            @pl.when(ch + 1 < NCHUNK)
            def _prefetch():
                @pl.when(ch >= 1)
                def _drain_prev():
                    for cp in out_copies(ch - 1, 1 - slot):
                        cp.wait()
                for cp in in_copies(ch + 1, 1 - slot):
                    cp.start()

            @pl.when(slot == 0)
            def _compute0():
                _chunk_compute(0, ch, wbuf, retbuf, knbuf, vnbuf, mkbuf, mvbuf)

            @pl.when(slot == 1)
            def _compute1():
                _chunk_compute(1, ch, wbuf, retbuf, knbuf, vnbuf, mkbuf, mvbuf)

            for cp in out_copies(ch, slot):
                cp.start()
            return carry1

        lax.fori_loop(0, NCHUNK, chunk_body, 0)
        for cp in out_copies(NCHUNK - 2, (NCHUNK - 2) % 2):
            cp.wait()
        for cp in out_copies(NCHUNK - 1, (NCHUNK - 1) % 2):
            cp.wait()
        or_cp.wait()
        return carry0

    lax.fori_loop(0, UPW, unit_body, 0)


def kernel(weights, key_new, value_new, reward, mem_keys, mem_values, mem_rewards):
    wT = weights.transpose(1, 0, 2)
    knT = key_new.transpose(1, 0, 2)
    vnT = value_new.transpose(1, 0, 2)
    rB = jnp.broadcast_to(reward.reshape(B, 1), (B, L))
    mk2 = mem_keys.reshape(U, PR, 128)
    mv2 = mem_values.reshape(U, PR, 128)
    mesh = plsc.VectorSubcoreMesh(core_axis_name="c", subcore_axis_name="s")
    sc = functools.partial(
        pl.kernel, mesh=mesh,
        out_type=[
            jax.ShapeDtypeStruct((U, PR, 128), jnp.float32),
            jax.ShapeDtypeStruct((U, PR, 128), jnp.float32),
            jax.ShapeDtypeStruct((U, C), jnp.float32),
        ],
        scratch_types=[
            pltpu.VMEM((B, C), jnp.float32),
            pltpu.VMEM((B, DK), jnp.float32),
            pltpu.VMEM((B, DV), jnp.float32),
            pltpu.VMEM((B, L), jnp.float32),
            pltpu.VMEM((C,), jnp.float32),
            pltpu.VMEM((C,), jnp.float32),
            pltpu.VMEM((2, CPR, 128), jnp.float32),
            pltpu.VMEM((2, CPR, 128), jnp.float32),
            pltpu.SemaphoreType.DMA((2, 2)),
            pltpu.SemaphoreType.DMA((2, 2)),
            pltpu.SemaphoreType.DMA,
        ],
    )(_sc_body)
    out_k, out_v, out_r = sc(wT, knT, vnT, rB, mk2, mv2, mem_rewards)
    return out_k.reshape(U, C, DK), out_v.reshape(U, C, DV), out_r


# speedup vs baseline: 2.1504x; 1.9878x over previous
import functools

import jax
import jax.numpy as jnp
from jax import lax
from jax.experimental import pallas as pl
from jax.experimental.pallas import tpu as pltpu
from jax.experimental.pallas import tpu_sc as plsc

B, U, C, DK, DV = 16, 64, 1024, 64, 64
RATE = 0.01
L = 16
NW = 32
UPW = U // NW
UB = 8


def _tc_body(w_ref, kn_ref, vn_ref, mk_ref, mv_ref, ok_ref, ov_ref):
    fresh = w_ref[...] * RATE
    retain = 1.0 - jnp.sum(fresh, axis=0)
    kv = jnp.concatenate([kn_ref[...], vn_ref[...]], axis=2)
    for i in range(UB):
        acc = jax.lax.dot_general(
            fresh[:, i, :], kv[:, i, :],
            dimension_numbers=(((0,), (0,)), ((), ())),
            preferred_element_type=jnp.float32)
        ok_ref[i] = mk_ref[i] * retain[i, :, None] + acc[:, :DK]
        ov_ref[i] = mv_ref[i] * retain[i, :, None] + acc[:, DK:]


def _sc_rewards(w_hbm, r_hbm, mr_hbm, or_hbm, wbuf, rbuf, orbuf):
    wid = lax.axis_index("s") * 2 + lax.axis_index("c")
    pltpu.sync_copy(r_hbm, rbuf)

    def unit_body(iu, carry0):
        u = wid * UPW + iu
        pltpu.sync_copy(w_hbm.at[u], wbuf)
        pltpu.sync_copy(mr_hbm.at[u], orbuf)
        rbv = [rbuf[b, :] for b in range(B)]

        def ret_body(g, carry):
            sl = pl.ds(g * L, L)
            wv0 = wbuf[0, sl]
            fs = wv0
            rw = wv0 * rbv[0]
            for b in range(1, B):
                wv = wbuf[b, sl]
                fs = fs + wv
                rw = rw + wv * rbv[b]
            orbuf[sl] = orbuf[sl] * (1.0 - RATE * fs) + RATE * rw
            return carry

        lax.fori_loop(0, C // L, ret_body, 0)
        pltpu.sync_copy(orbuf, or_hbm.at[u])
        return carry0

    lax.fori_loop(0, UPW, unit_body, 0)


def kernel(weights, key_new, value_new, reward, mem_keys, mem_values, mem_rewards):
    out_k, out_v = pl.pallas_call(
        _tc_body,
        grid=(U // UB,),
        in_specs=[
            pl.BlockSpec((B, UB, C), lambda u: (0, u, 0)),
            pl.BlockSpec((B, UB, DK), lambda u: (0, u, 0)),
            pl.BlockSpec((B, UB, DV), lambda u: (0, u, 0)),
            pl.BlockSpec((UB, C, DK), lambda u: (u, 0, 0)),
            pl.BlockSpec((UB, C, DV), lambda u: (u, 0, 0)),
        ],
        out_specs=[
            pl.BlockSpec((UB, C, DK), lambda u: (u, 0, 0)),
            pl.BlockSpec((UB, C, DV), lambda u: (u, 0, 0)),
        ],
        out_shape=[
            jax.ShapeDtypeStruct((U, C, DK), jnp.float32),
            jax.ShapeDtypeStruct((U, C, DV), jnp.float32),
        ],
    )(weights, key_new, value_new, mem_keys, mem_values)

    wT = weights.transpose(1, 0, 2)
    rB = jnp.broadcast_to(reward.reshape(B, 1), (B, L))
    mesh = plsc.VectorSubcoreMesh(core_axis_name="c", subcore_axis_name="s")
    out_r = functools.partial(
        pl.kernel, mesh=mesh,
        out_type=jax.ShapeDtypeStruct((U, C), jnp.float32),
        scratch_types=[
            pltpu.VMEM((B, C), jnp.float32),
            pltpu.VMEM((B, L), jnp.float32),
            pltpu.VMEM((C,), jnp.float32),
        ],
    )(_sc_rewards)(wT, rB, mem_rewards)
    return out_k, out_v, out_r
